# baseline (speedup 1.0000x reference)
import functools

import jax
import jax.numpy as jnp
from jax import lax
from jax.experimental import pallas as pl
from jax.experimental.pallas import tpu as pltpu
from jax.experimental.pallas import tpu_sc as plsc

_ROWS, _COLS = 16384, 4096
_BLOCK_ROWS = 256
_CHUNK_ROWS = 32
_N_IDX = 2048
_N_SUBCORES = 16
_IDX_PER_SUB = _N_IDX // _N_SUBCORES
_COLS_PER_SUB = _COLS // _N_SUBCORES


def _sc_mask_kernel(idx_hbm, zeros_hbm, ones_hbm, mask_hbm,
                    idx_v, ones_v, shared):
    s = lax.axis_index("s")
    pltpu.sync_copy(zeros_hbm.at[pl.ds(s * _COLS_PER_SUB, _COLS_PER_SUB)],
                    shared.at[pl.ds(s * _COLS_PER_SUB, _COLS_PER_SUB)])
    pltpu.sync_copy(idx_hbm.at[pl.ds(s * _IDX_PER_SUB, _IDX_PER_SUB)], idx_v)
    pltpu.sync_copy(ones_hbm, ones_v)
    plsc.subcore_barrier()
    pltpu.sync_copy(ones_v, shared.at[idx_v], add=True)
    plsc.subcore_barrier()
    pltpu.sync_copy(shared.at[pl.ds(s * _COLS_PER_SUB, _COLS_PER_SUB)],
                    mask_hbm.at[pl.ds(s * _COLS_PER_SUB, _COLS_PER_SUB)])


def _sc_mask(indices, zeros, ones):
    mesh = plsc.VectorSubcoreMesh(
        core_axis_name="c", subcore_axis_name="s", num_cores=1)
    return pl.kernel(
        _sc_mask_kernel,
        out_type=jax.ShapeDtypeStruct((_COLS,), jnp.float32),
        mesh=mesh,
        scratch_types=[
            pltpu.VMEM((_IDX_PER_SUB,), jnp.int32),
            pltpu.VMEM((_IDX_PER_SUB,), jnp.float32),
            pltpu.VMEM_SHARED((_COLS,), jnp.float32),
        ],
    )(indices, zeros, ones)


def _select_kernel(x_ref, p_ref, mask_ref, o_ref):
    t = p_ref[0, 0]
    m = mask_ref[...] != 0.0

    def row_body(r, carry):
        xb = x_ref[pl.ds(r * _CHUNK_ROWS, _CHUNK_ROWS), :]
        o_ref[pl.ds(r * _CHUNK_ROWS, _CHUNK_ROWS), :] = jnp.where(
            m, (xb > t).astype(xb.dtype), xb)
        return carry

    jax.lax.fori_loop(0, _BLOCK_ROWS // _CHUNK_ROWS, row_body, 0)


@functools.partial(jax.jit, static_argnames=())
def kernel(x, params, indices):
    p2 = params.reshape(1, 1)
    zeros = jnp.zeros((_COLS,), jnp.float32)
    ones = jnp.ones((_IDX_PER_SUB,), jnp.float32)
    mask = _sc_mask(indices, zeros, ones).reshape(1, _COLS)

    grid = _ROWS // _BLOCK_ROWS
    return pl.pallas_call(
        _select_kernel,
        grid=(grid,),
        in_specs=[
            pl.BlockSpec((_BLOCK_ROWS, _COLS), lambda i: (i, 0)),
            pl.BlockSpec((1, 1), lambda i: (0, 0)),
            pl.BlockSpec((1, _COLS), lambda i: (0, 0)),
        ],
        out_specs=pl.BlockSpec((_BLOCK_ROWS, _COLS), lambda i: (i, 0)),
        out_shape=jax.ShapeDtypeStruct((_ROWS, _COLS), x.dtype),
    )(x, p2, mask)


# speedup vs baseline: 1.0133x; 1.0133x over previous
import functools

import jax
import jax.numpy as jnp
from jax import lax
from jax.experimental import pallas as pl
from jax.experimental.pallas import tpu as pltpu
from jax.experimental.pallas import tpu_sc as plsc

_ROWS, _COLS = 16384, 4096
_BLOCK_ROWS = 512
_CHUNK_ROWS = 32
_N_IDX = 2048
_N_SUBCORES = 16
_IDX_PER_SUB = _N_IDX // _N_SUBCORES
_COLS_PER_SUB = _COLS // _N_SUBCORES


def _sc_mask_kernel(idx_hbm, zeros_hbm, ones_hbm, mask_hbm,
                    idx_v, ones_v, shared):
    s = lax.axis_index("s")
    pltpu.sync_copy(zeros_hbm.at[pl.ds(s * _COLS_PER_SUB, _COLS_PER_SUB)],
                    shared.at[pl.ds(s * _COLS_PER_SUB, _COLS_PER_SUB)])
    pltpu.sync_copy(idx_hbm.at[pl.ds(s * _IDX_PER_SUB, _IDX_PER_SUB)], idx_v)
    pltpu.sync_copy(ones_hbm, ones_v)
    plsc.subcore_barrier()
    pltpu.sync_copy(ones_v, shared.at[idx_v], add=True)
    plsc.subcore_barrier()
    pltpu.sync_copy(shared.at[pl.ds(s * _COLS_PER_SUB, _COLS_PER_SUB)],
                    mask_hbm.at[pl.ds(s * _COLS_PER_SUB, _COLS_PER_SUB)])


def _sc_mask(indices, zeros, ones):
    mesh = plsc.VectorSubcoreMesh(
        core_axis_name="c", subcore_axis_name="s", num_cores=1)
    return pl.kernel(
        _sc_mask_kernel,
        out_type=jax.ShapeDtypeStruct((_COLS,), jnp.float32),
        mesh=mesh,
        scratch_types=[
            pltpu.VMEM((_IDX_PER_SUB,), jnp.int32),
            pltpu.VMEM((_IDX_PER_SUB,), jnp.float32),
            pltpu.VMEM_SHARED((_COLS,), jnp.float32),
        ],
    )(indices, zeros, ones)


def _select_kernel(x_ref, p_ref, mask_ref, o_ref):
    t = p_ref[0, 0]
    m = mask_ref[...] != 0.0

    def row_body(r, carry):
        xb = x_ref[pl.ds(r * _CHUNK_ROWS, _CHUNK_ROWS), :]
        o_ref[pl.ds(r * _CHUNK_ROWS, _CHUNK_ROWS), :] = jnp.where(
            m, (xb > t).astype(xb.dtype), xb)
        return carry

    jax.lax.fori_loop(0, _BLOCK_ROWS // _CHUNK_ROWS, row_body, 0)


@functools.partial(jax.jit, static_argnames=())
def kernel(x, params, indices):
    p2 = params.reshape(1, 1)
    zeros = jnp.zeros((_COLS,), jnp.float32)
    ones = jnp.ones((_IDX_PER_SUB,), jnp.float32)
    mask = _sc_mask(indices, zeros, ones).reshape(1, _COLS)

    grid = _ROWS // _BLOCK_ROWS
    return pl.pallas_call(
        _select_kernel,
        grid=(grid,),
        in_specs=[
            pl.BlockSpec((_BLOCK_ROWS, _COLS), lambda i: (i, 0)),
            pl.BlockSpec((1, 1), lambda i: (0, 0)),
            pl.BlockSpec((1, _COLS), lambda i: (0, 0)),
        ],
        out_specs=pl.BlockSpec((_BLOCK_ROWS, _COLS), lambda i: (i, 0)),
        out_shape=jax.ShapeDtypeStruct((_ROWS, _COLS), x.dtype),
        compiler_params=pltpu.CompilerParams(
            dimension_semantics=("parallel",)),
    )(x, p2, mask)


# speedup vs baseline: 1.0140x; 1.0007x over previous
import functools

import jax
import jax.numpy as jnp
from jax import lax
from jax.experimental import pallas as pl
from jax.experimental.pallas import tpu as pltpu
from jax.experimental.pallas import tpu_sc as plsc

_ROWS, _COLS = 16384, 4096
_BLOCK_ROWS = 512
_CHUNK_ROWS = 32
_N_IDX = 2048
_N_SUBCORES = 16
_IDX_PER_SUB = _N_IDX // _N_SUBCORES
_COLS_PER_SUB = _COLS // _N_SUBCORES


def _sc_mask_kernel(idx_hbm, zeros_hbm, ones_hbm, mask_hbm,
                    idx_v, ones_v, shared):
    s = lax.axis_index("s")
    pltpu.sync_copy(zeros_hbm.at[pl.ds(s * _COLS_PER_SUB, _COLS_PER_SUB)],
                    shared.at[pl.ds(s * _COLS_PER_SUB, _COLS_PER_SUB)])
    pltpu.sync_copy(idx_hbm.at[pl.ds(s * _IDX_PER_SUB, _IDX_PER_SUB)], idx_v)
    pltpu.sync_copy(ones_hbm, ones_v)
    plsc.subcore_barrier()
    pltpu.sync_copy(ones_v, shared.at[idx_v], add=True)
    plsc.subcore_barrier()
    pltpu.sync_copy(shared.at[pl.ds(s * _COLS_PER_SUB, _COLS_PER_SUB)],
                    mask_hbm.at[pl.ds(s * _COLS_PER_SUB, _COLS_PER_SUB)])


def _sc_mask(indices, zeros, ones):
    mesh = plsc.VectorSubcoreMesh(
        core_axis_name="c", subcore_axis_name="s", num_cores=1)
    return pl.kernel(
        _sc_mask_kernel,
        out_type=jax.ShapeDtypeStruct((_COLS,), jnp.float32),
        mesh=mesh,
        scratch_types=[
            pltpu.VMEM((_IDX_PER_SUB,), jnp.int32),
            pltpu.VMEM((_IDX_PER_SUB,), jnp.float32),
            pltpu.VMEM_SHARED((_COLS,), jnp.float32),
        ],
    )(indices, zeros, ones)


def _select_kernel(x_ref, p_ref, mask_ref, o_ref):
    t = p_ref[0, 0]
    m = mask_ref[...] != 0.0

    def row_body(r, carry):
        xb = x_ref[pl.ds(r * _CHUNK_ROWS, _CHUNK_ROWS), :]
        o_ref[pl.ds(r * _CHUNK_ROWS, _CHUNK_ROWS), :] = jnp.where(
            m, (xb > t).astype(xb.dtype), xb)
        return carry

    jax.lax.fori_loop(0, _BLOCK_ROWS // _CHUNK_ROWS, row_body, 0)


@functools.partial(jax.jit, static_argnames=())
def kernel(x, params, indices):
    p2 = params.reshape(1, 1)
    zeros = jnp.zeros((_COLS,), jnp.float32)
    ones = jnp.ones((_IDX_PER_SUB,), jnp.float32)
    mask = _sc_mask(indices, zeros, ones).reshape(1, _COLS)

    grid = _ROWS // _BLOCK_ROWS
    return pl.pallas_call(
        _select_kernel,
        grid=(grid,),
        in_specs=[
            pl.BlockSpec((_BLOCK_ROWS, _COLS), lambda i: (i, 0)),
            pl.BlockSpec((1, 1), lambda i: (0, 0)),
            pl.BlockSpec((1, _COLS), lambda i: (0, 0)),
        ],
        out_specs=pl.BlockSpec((_BLOCK_ROWS, _COLS), lambda i: (i, 0)),
        out_shape=jax.ShapeDtypeStruct((_ROWS, _COLS), x.dtype),
    )(x, p2, mask)
